# baseline (speedup 1.0000x reference)
import functools

import jax
import jax.numpy as jnp
from jax import lax
from jax.experimental import pallas as pl
from jax.experimental.pallas import tpu as pltpu
from jax.experimental.pallas import tpu_sc as plsc

_USERS_BUCKETS = 1000000
_MOVIES_BUCKETS = 100000
_D = 32
_B = 16384
_NW = 32
_BPW = _B // _NW
_L = 16
_BLK = 128
_NBLK = _BPW // _BLK
_NCK = _BPW // _L

_FT_U = _USERS_BUCKETS // 128 * 128
_FT_M = _MOVIES_BUCKETS // 128 * 128
_CHUNK_U = 8 * _FT_U
_CHUNK_M = 8 * _FT_M


def _vec_mod(v, n):
  q = (v.astype(jnp.float32) * (1.0 / n)).astype(jnp.int32)
  r = v - q * n
  r = jnp.where(r >= n, r - n, r)
  r = jnp.where(r < 0, r + n, r)
  return r


def _chunk_flat(table, a, ft):
  t = table[:ft, 8 * a:8 * a + 8].T
  return t.reshape(8, ft // 128, 128).transpose(1, 0, 2).reshape(-1)


def _lw_flat(table, ft):
  t = table[ft:, :].T
  t = jnp.pad(t, ((0, 0), (0, 128 - t.shape[1])))
  return t.reshape(4, 8, 1, 128).transpose(0, 2, 1, 3).reshape(-1)


def _body(uids_hbm, mids_hbm,
          u0_hbm, u1_hbm, u2_hbm, u3_hbm, ulw_hbm,
          m0_hbm, m1_hbm, m2_hbm, m3_hbm, mlw_hbm,
          out_hbm,
          uids_v, mids_v, idxu_v, idxm_v, lwdu_v, lwdm_v,
          uvals_v, mvals_v, ulw_v, mlw_v, out_v, sem):
  wid = lax.axis_index("s") * 2 + lax.axis_index("c")
  base = wid * _BPW
  uchunks = (u0_hbm, u1_hbm, u2_hbm, u3_hbm)
  mchunks = (m0_hbm, m1_hbm, m2_hbm, m3_hbm)

  cps = [pltpu.async_copy(uids_hbm.at[pl.ds(base, _BPW)], uids_v, sem),
         pltpu.async_copy(mids_hbm.at[pl.ds(base, _BPW)], mids_v, sem),
         pltpu.async_copy(ulw_hbm, ulw_v, sem),
         pltpu.async_copy(mlw_hbm, mlw_v, sem)]
  for cp in cps:
    cp.wait()

  for l in range(_NCK):
    sl = pl.ds(l * _L, _L)
    ru = _vec_mod(uids_v[sl], _USERS_BUCKETS)
    rm = _vec_mod(mids_v[sl], _MOVIES_BUCKETS)
    lwdu_v[sl] = ru - _FT_U
    lwdm_v[sl] = rm - _FT_M
    rcu = jnp.minimum(ru, _FT_U - 1)
    rcm = jnp.minimum(rm, _FT_M - 1)
    idxu_v[sl] = ((rcu >> 7) << 10) + (rcu & 127)
    idxm_v[sl] = ((rcm >> 7) << 10) + (rcm & 127)

  copies = []
  for kb in range(_NBLK):
    isl = pl.ds(kb * _BLK, _BLK)
    for j in range(_D):
      off = (j % 8) * 128
      vsl = pl.ds((kb * _D + j) * _BLK, _BLK)
      copies.append(pltpu.async_copy(
          uchunks[j // 8].at[pl.ds(off, _CHUNK_U - off)].at[idxu_v.at[isl]],
          uvals_v.at[vsl], sem))
      copies.append(pltpu.async_copy(
          mchunks[j // 8].at[pl.ds(off, _CHUNK_M - off)].at[idxm_v.at[isl]],
          mvals_v.at[vsl], sem))
  for cp in copies:
    cp.wait()

  lane = lax.iota(jnp.int32, _L)

  def patch_body(l, carry):
    kb = l // (_BLK // _L)
    lb = l % (_BLK // _L)
    sl = pl.ds(l * _L, _L)
    du = lwdu_v[sl]
    dm = lwdm_v[sl]

    @pl.when(lax.reduce_max(du, axes=(0,)) >= 0)
    def _():
      m = du >= 0
      for j in range(_D):
        addr = (j // 8) * 1024 + (j % 8) * 128 + du
        patched = plsc.load_gather(ulw_v, [addr], mask=m)
        vpos = (kb * _D + j) * _BLK + lb * _L + lane
        plsc.store_scatter(uvals_v, [vpos], patched, mask=m)

    @pl.when(lax.reduce_max(dm, axes=(0,)) >= 0)
    def _():
      m = dm >= 0
      for j in range(_D):
        addr = (j // 8) * 1024 + (j % 8) * 128 + dm
        patched = plsc.load_gather(mlw_v, [addr], mask=m)
        vpos = (kb * _D + j) * _BLK + lb * _L + lane
        plsc.store_scatter(mvals_v, [vpos], patched, mask=m)

    return carry

  lax.fori_loop(0, _NCK, patch_body, 0)

  def block_body(kb, carry):
    vbase = kb * _D * _BLK
    for l in range(_BLK // _L):
      acc = jnp.zeros((_L,), jnp.float32)
      for j in range(_D):
        sl = pl.ds(vbase + j * _BLK + l * _L, _L)
        acc = acc + uvals_v[sl] * mvals_v[sl]
      out_v[pl.ds(kb * _BLK + l * _L, _L)] = 1.0 / (1.0 + jnp.exp(-acc))
    return carry

  lax.fori_loop(0, _NBLK, block_body, 0)

  pltpu.sync_copy(out_v, out_hbm.at[pl.ds(base, _BPW)])


def kernel(users_ids, movies_ids, user_table, movie_table):
  mesh = plsc.VectorSubcoreMesh(core_axis_name="c", subcore_axis_name="s")
  run = functools.partial(
      pl.kernel,
      mesh=mesh,
      compiler_params=pltpu.CompilerParams(
          needs_layout_passes=False, use_tc_tiling_on_sc=False,
          disable_bounds_checks=True),
      out_type=jax.ShapeDtypeStruct((_B,), jnp.float32),
      scratch_types=[
          pltpu.VMEM((_BPW,), jnp.int32),
          pltpu.VMEM((_BPW,), jnp.int32),
          pltpu.VMEM((_BPW,), jnp.int32),
          pltpu.VMEM((_BPW,), jnp.int32),
          pltpu.VMEM((_BPW,), jnp.int32),
          pltpu.VMEM((_BPW,), jnp.int32),
          pltpu.VMEM((_D * _BPW,), jnp.float32),
          pltpu.VMEM((_D * _BPW,), jnp.float32),
          pltpu.VMEM((4096,), jnp.float32),
          pltpu.VMEM((4096,), jnp.float32),
          pltpu.VMEM((_BPW,), jnp.float32),
          pltpu.SemaphoreType.DMA,
      ],
  )(_body)
  return run(users_ids, movies_ids,
             _chunk_flat(user_table, 0, _FT_U),
             _chunk_flat(user_table, 1, _FT_U),
             _chunk_flat(user_table, 2, _FT_U),
             _chunk_flat(user_table, 3, _FT_U),
             _lw_flat(user_table, _FT_U),
             _chunk_flat(movie_table, 0, _FT_M),
             _chunk_flat(movie_table, 1, _FT_M),
             _chunk_flat(movie_table, 2, _FT_M),
             _chunk_flat(movie_table, 3, _FT_M),
             _lw_flat(movie_table, _FT_M))
